# baseline (speedup 1.0000x reference)
import functools

import jax
import jax.numpy as jnp
from jax.experimental import pallas as pl
from jax.experimental.pallas import tpu as pltpu

HID = 128
F = 5

_BM = 80


def _fused_kernel(x_ref, w1_ref, b1_ref, w2_ref, b2_ref, wf_ref, bf_ref,
                  wx_ref, bx_ref, lam_ref, gram_ref, res_ref, score_ref,
                  resb_ref):
    i = pl.program_id(0)

    @pl.when(i == 0)
    def _front():
        x = x_ref[...]
        h = jnp.maximum(
            jax.lax.dot_general(x, w1_ref[...], (((1,), (1,)), ((), ())),
                                preferred_element_type=jnp.float32)
            + b1_ref[...], 0.0)
        h = jax.lax.dot_general(h, w2_ref[...], (((1,), (1,)), ((), ())),
                                preferred_element_type=jnp.float32) + b2_ref[...]
        g = jax.lax.dot_general(h, wf_ref[...], (((1,), (1,)), ((), ())),
                                preferred_element_type=jnp.float32)
        xp = jnp.tanh(
            jax.lax.dot_general(h, wx_ref[...], (((1,), (1,)), ((), ())),
                                preferred_element_type=jnp.float32)
            + bx_ref[...])
        lams = jax.nn.softmax(lam_ref[...], axis=-1)
        logits = []
        for k in range(F):
            hp = jnp.tanh(lams[0, k] * g + bf_ref[...])
            logits.append(jnp.sum(hp * xp, axis=1, keepdims=True))
        logit = jnp.concatenate(logits, axis=1)
        m = jnp.max(logit, axis=1, keepdims=True)
        e = jnp.exp(logit - m)
        score = e / jnp.sum(e, axis=1, keepdims=True)
        s = jnp.sum(score * lams, axis=1, keepdims=True)
        r = h * s
        res_ref[...] = r
        score_ref[...] = score
        resb_ref[...] = r.astype(jnp.bfloat16)

    a = resb_ref[pl.ds(i * _BM, _BM), :]
    gram_ref[...] = jax.lax.dot_general(
        a, resb_ref[...], (((1,), (1,)), ((), ())),
        preferred_element_type=jnp.float32)


@functools.partial(jax.jit, static_argnames=())
def kernel(x, edge_index, lin1_w, lin1_b, lin2_w, lin2_b, filter_weights,
           wf_w, wf_b, wx_w, wx_b, lam):
    n = x.shape[0]
    b1 = lin1_b.reshape(1, HID)
    b2 = lin2_b.reshape(1, HID)
    bf = wf_b.reshape(1, HID)
    bx = wx_b.reshape(1, HID)
    lam2 = lam.reshape(1, F)

    full = lambda shp: pl.BlockSpec(shp, lambda i: (0, 0))
    res_, res, score = pl.pallas_call(
        _fused_kernel,
        grid=(n // _BM,),
        in_specs=[
            full((n, HID)),
            full((HID, HID)), full((1, HID)),
            full((HID, HID)), full((1, HID)),
            full((HID, HID)), full((1, HID)),
            full((HID, HID)), full((1, HID)),
            full((1, F)),
        ],
        out_specs=[
            pl.BlockSpec((_BM, n), lambda i: (i, 0)),
            full((n, HID)),
            full((n, F)),
        ],
        out_shape=[
            jax.ShapeDtypeStruct((n, n), jnp.float32),
            jax.ShapeDtypeStruct((n, HID), jnp.float32),
            jax.ShapeDtypeStruct((n, F), jnp.float32),
        ],
        scratch_shapes=[pltpu.VMEM((n, HID), jnp.bfloat16)],
    )(x, lin1_w, b1, lin2_w, b2, wf_w, bf, wx_w, bx, lam2)

    return (res_, res, score.T)


# speedup vs baseline: 1.3186x; 1.3186x over previous
import functools

import jax
import jax.numpy as jnp
from jax.experimental import pallas as pl
from jax.experimental.pallas import tpu as pltpu

HID = 128
F = 5

_BM = 200
_FRONT_BLK = 2000


def _fused_kernel(x_ref, w1_ref, b1_ref, w2_ref, b2_ref, wf_ref, bf_ref,
                  wx_ref, bx_ref, lam_ref, gram_ref, res_ref, score_ref,
                  resb_ref):
    i = pl.program_id(0)

    @pl.when(i == 0)
    def _front():
        lams = jax.nn.softmax(lam_ref[...], axis=-1)
        for c in range(0, x_ref.shape[0], _FRONT_BLK):
            x = x_ref[c:c + _FRONT_BLK, :]
            h = jnp.maximum(
                jax.lax.dot_general(x, w1_ref[...], (((1,), (1,)), ((), ())),
                                    preferred_element_type=jnp.float32)
                + b1_ref[...], 0.0)
            h = jax.lax.dot_general(h, w2_ref[...], (((1,), (1,)), ((), ())),
                                    preferred_element_type=jnp.float32) + b2_ref[...]
            g = jax.lax.dot_general(h, wf_ref[...], (((1,), (1,)), ((), ())),
                                    preferred_element_type=jnp.float32)
            xp = jnp.tanh(
                jax.lax.dot_general(h, wx_ref[...], (((1,), (1,)), ((), ())),
                                    preferred_element_type=jnp.float32)
                + bx_ref[...])
            logits = []
            for k in range(F):
                hp = jnp.tanh(lams[0, k] * g + bf_ref[...])
                logits.append(jnp.sum(hp * xp, axis=1, keepdims=True))
            logit = jnp.concatenate(logits, axis=1)
            m = jnp.max(logit, axis=1, keepdims=True)
            e = jnp.exp(logit - m)
            score = e / jnp.sum(e, axis=1, keepdims=True)
            s = jnp.sum(score * lams, axis=1, keepdims=True)
            r = h * s
            res_ref[c:c + _FRONT_BLK, :] = r
            score_ref[c:c + _FRONT_BLK, :] = score
            resb_ref[c:c + _FRONT_BLK, :] = r.astype(jnp.bfloat16)

    a = resb_ref[pl.ds(i * _BM, _BM), :]
    gram_ref[...] = jax.lax.dot_general(
        a, resb_ref[...], (((1,), (1,)), ((), ())),
        preferred_element_type=jnp.float32)


@functools.partial(jax.jit, static_argnames=())
def kernel(x, edge_index, lin1_w, lin1_b, lin2_w, lin2_b, filter_weights,
           wf_w, wf_b, wx_w, wx_b, lam):
    n = x.shape[0]
    b1 = lin1_b.reshape(1, HID)
    b2 = lin2_b.reshape(1, HID)
    bf = wf_b.reshape(1, HID)
    bx = wx_b.reshape(1, HID)
    lam2 = lam.reshape(1, F)

    full = lambda shp: pl.BlockSpec(shp, lambda i: (0, 0))
    res_, res, score = pl.pallas_call(
        _fused_kernel,
        grid=(n // _BM,),
        in_specs=[
            full((n, HID)),
            full((HID, HID)), full((1, HID)),
            full((HID, HID)), full((1, HID)),
            full((HID, HID)), full((1, HID)),
            full((HID, HID)), full((1, HID)),
            full((1, F)),
        ],
        out_specs=[
            pl.BlockSpec((_BM, n), lambda i: (i, 0)),
            full((n, HID)),
            full((n, F)),
        ],
        out_shape=[
            jax.ShapeDtypeStruct((n, n), jnp.float32),
            jax.ShapeDtypeStruct((n, HID), jnp.float32),
            jax.ShapeDtypeStruct((n, F), jnp.float32),
        ],
        scratch_shapes=[pltpu.VMEM((n, HID), jnp.bfloat16)],
    )(x, lin1_w, b1, lin2_w, b2, wf_w, bf, wx_w, bx, lam2)

    return (res_, res, score.T)


# speedup vs baseline: 1.3237x; 1.0039x over previous
import functools

import jax
import jax.numpy as jnp
from jax.experimental import pallas as pl
from jax.experimental.pallas import tpu as pltpu

HID = 128
F = 5

_BM = 200
_FRONT_BLK = 2000


def _fused_kernel(x_ref, w1_ref, b1_ref, w2_ref, b2_ref, wf_ref, bf_ref,
                  wx_ref, bx_ref, lam_ref, gram_ref, res_ref, score_ref,
                  resb_ref):
    i = pl.program_id(0)

    @pl.when(i == 0)
    def _front():
        lams = jax.nn.softmax(lam_ref[...], axis=-1)
        for c in range(0, x_ref.shape[0], _FRONT_BLK):
            x = x_ref[c:c + _FRONT_BLK, :]
            h = jnp.maximum(
                jax.lax.dot_general(x, w1_ref[...], (((1,), (1,)), ((), ())),
                                    preferred_element_type=jnp.float32)
                + b1_ref[...], 0.0)
            h = jax.lax.dot_general(h, w2_ref[...], (((1,), (1,)), ((), ())),
                                    preferred_element_type=jnp.float32) + b2_ref[...]
            g = jax.lax.dot_general(h, wf_ref[...], (((1,), (1,)), ((), ())),
                                    preferred_element_type=jnp.float32)
            xp = jnp.tanh(
                jax.lax.dot_general(h, wx_ref[...], (((1,), (1,)), ((), ())),
                                    preferred_element_type=jnp.float32)
                + bx_ref[...])
            logits = []
            for k in range(F):
                hp = jnp.tanh(lams[0, k] * g + bf_ref[...])
                logits.append(jnp.sum(hp * xp, axis=1, keepdims=True))
            logit = jnp.concatenate(logits, axis=1)
            m = jnp.max(logit, axis=1, keepdims=True)
            e = jnp.exp(logit - m)
            score = e / jnp.sum(e, axis=1, keepdims=True)
            s = jnp.sum(score * lams, axis=1, keepdims=True)
            r = h * s
            res_ref[c:c + _FRONT_BLK, :] = r
            score_ref[c:c + _FRONT_BLK, :] = score
            resb_ref[c:c + _FRONT_BLK, :] = r.astype(jnp.bfloat16)

    a = res_ref[pl.ds(i * _BM, _BM), :].astype(jnp.bfloat16)
    gram_ref[...] = jax.lax.dot_general(
        a, resb_ref[...], (((1,), (1,)), ((), ())),
        preferred_element_type=jnp.float32)


@functools.partial(jax.jit, static_argnames=())
def kernel(x, edge_index, lin1_w, lin1_b, lin2_w, lin2_b, filter_weights,
           wf_w, wf_b, wx_w, wx_b, lam):
    n = x.shape[0]
    b1 = lin1_b.reshape(1, HID)
    b2 = lin2_b.reshape(1, HID)
    bf = wf_b.reshape(1, HID)
    bx = wx_b.reshape(1, HID)
    lam2 = lam.reshape(1, F)

    full = lambda shp: pl.BlockSpec(shp, lambda i: (0, 0))
    res_, res, score = pl.pallas_call(
        _fused_kernel,
        grid=(n // _BM,),
        in_specs=[
            full((n, HID)),
            full((HID, HID)), full((1, HID)),
            full((HID, HID)), full((1, HID)),
            full((HID, HID)), full((1, HID)),
            full((HID, HID)), full((1, HID)),
            full((1, F)),
        ],
        out_specs=[
            pl.BlockSpec((_BM, n), lambda i: (i, 0)),
            full((n, HID)),
            full((n, F)),
        ],
        out_shape=[
            jax.ShapeDtypeStruct((n, n), jnp.float32),
            jax.ShapeDtypeStruct((n, HID), jnp.float32),
            jax.ShapeDtypeStruct((n, F), jnp.float32),
        ],
        scratch_shapes=[pltpu.VMEM((n, HID), jnp.bfloat16)],
    )(x, lin1_w, b1, lin2_w, b2, wf_w, bf, wx_w, bx, lam2)

    return (res_, res, score.T)
